# baseline (speedup 1.0000x reference)
import functools

import jax
import jax.numpy as jnp
from jax import lax
from jax.experimental import pallas as pl
from jax.experimental.pallas import tpu as pltpu
from jax.experimental.pallas import tpu_sc as plsc

TM = 128
TBR = 512
CH_S = 16
NC = 2
NWORK = 32


def _softplus(x):
    return jnp.maximum(x, 0.0) + jnp.log1p(jnp.exp(-jnp.abs(x)))


def _gelu(x):
    return x * 0.5 * (1.0 + lax.erf(x * 0.7071067811865476))


def _router_body(x_ref, wgn_ref, bgn_ref, epsT_ref, eid_ref, gate_ref):
    nexp = epsT_ref.shape[0]
    logits2 = lax.dot_general(x_ref[...], wgn_ref[...],
                              (((1,), (0,)), ((), ())),
                              preferred_element_type=jnp.float32)
    l2t = logits2.T + bgn_ref[...]
    lg = l2t[:nexp]
    nl = l2t[nexp:]
    noisy = lg + epsT_ref[...] * _softplus(nl)
    iota = lax.broadcasted_iota(jnp.int32, noisy.shape, 0)
    m1 = jnp.max(noisy, axis=0, keepdims=True)
    a1 = jnp.min(jnp.where(noisy == m1, iota, nexp),
                 axis=0, keepdims=True)
    masked = jnp.where(iota == a1, -jnp.inf, noisy)
    m2 = jnp.max(masked, axis=0, keepdims=True)
    a2 = jnp.min(jnp.where(masked == m2, iota, nexp),
                 axis=0, keepdims=True)
    t = jnp.exp(m2 - m1)
    g1 = 1.0 / (1.0 + t)
    g2 = t * g1
    eid_ref[...] = jnp.concatenate([a1, a2], axis=0).astype(jnp.float32)
    nl16 = gate_ref.shape[2]
    g1b = jnp.broadcast_to(g1[:, :, None], (1, g1.shape[1], nl16))
    g2b = jnp.broadcast_to(g2[:, :, None], (1, g2.shape[1], nl16))
    gate_ref[...] = jnp.concatenate([g1b, g2b], axis=0)


def _router(flat_x, wgn, bgn_col, epsT):
    t, d = flat_x.shape
    nexp = epsT.shape[0]
    grid = (t // TBR,)
    return pl.pallas_call(
        _router_body,
        grid=grid,
        in_specs=[
            pl.BlockSpec((TBR, d), lambda i: (i, 0)),
            pl.BlockSpec((d, 2 * nexp), lambda i: (0, 0)),
            pl.BlockSpec((2 * nexp, 1), lambda i: (0, 0)),
            pl.BlockSpec((nexp, TBR), lambda i: (0, i)),
        ],
        out_specs=[
            pl.BlockSpec((2, TBR), lambda i: (0, i)),
            pl.BlockSpec((2, TBR, 16), lambda i: (0, i, 0)),
        ],
        out_shape=[
            jax.ShapeDtypeStruct((2, t), jnp.float32),
            jax.ShapeDtypeStruct((2, t, 16), jnp.float32),
        ],
        compiler_params=pltpu.CompilerParams(
            dimension_semantics=("parallel",)),
    )(flat_x, wgn, bgn_col, epsT)


def _dispatch_body(nexp, ntc, e3_ref, dest_ref, eot_ref):
    e3 = e3_ref[...].astype(jnp.int32)
    rj, cj = e3.shape
    eio = lax.broadcasted_iota(jnp.int32, (rj, nexp, cj), 1)
    x3 = (e3[:, None, :] == eio).astype(jnp.float32)
    x2 = x3.reshape(rj * nexp, cj)
    ci = lax.broadcasted_iota(jnp.int32, (cj, cj), 0)
    cc = lax.broadcasted_iota(jnp.int32, (cj, cj), 1)
    ltc = (ci <= cc).astype(jnp.float32)
    p2 = lax.dot_general(x2, ltc, (((1,), (0,)), ((), ())),
                         preferred_element_type=jnp.float32)
    p3 = p2.reshape(rj, nexp, cj)
    within = jnp.sum(p3 * x3, axis=1)
    bt = jnp.sum(x3, axis=2)
    ri = lax.broadcasted_iota(jnp.int32, (rj, rj), 0)
    rr = lax.broadcasted_iota(jnp.int32, (rj, rj), 1)
    sl = (rr < ri).astype(jnp.float32)
    exb = lax.dot_general(sl, bt, (((1,), (0,)), ((), ())),
                          preferred_element_type=jnp.float32)
    counts = jnp.sum(bt, axis=0, keepdims=True)
    pe = jnp.ceil(counts / TM) * TM
    ei = lax.broadcasted_iota(jnp.int32, (nexp, nexp), 0)
    ee = lax.broadcasted_iota(jnp.int32, (nexp, nexp), 1)
    sle = (ei < ee).astype(jnp.float32)
    ss = lax.dot_general(pe, sle, (((1,), (0,)), ((), ())),
                         preferred_element_type=jnp.float32)
    base = jnp.sum((exb[:, :, None] + ss[:, :, None]) * x3, axis=1)
    dest_ref[...] = (base + within - 1.0).astype(jnp.int32)
    ti = (lax.broadcasted_iota(jnp.int32, (nexp, ntc), 1) * TM
          ).astype(jnp.float32)
    eot = jnp.sum((ss.T <= ti).astype(jnp.float32), axis=0,
                  keepdims=True) - 1.0
    eot_ref[...] = eot.astype(jnp.int32)


def _dispatch(e3, nexp, ntc):
    rj, cj = e3.shape
    return pl.pallas_call(
        functools.partial(_dispatch_body, nexp, ntc),
        grid=(1,),
        in_specs=[pl.BlockSpec((rj, cj), lambda i: (0, 0))],
        out_specs=[
            pl.BlockSpec((rj, cj), lambda i: (0, 0)),
            pl.BlockSpec((1, ntc), lambda i: (0, 0)),
        ],
        out_shape=[
            jax.ShapeDtypeStruct((rj, cj), jnp.int32),
            jax.ShapeDtypeStruct((1, ntc), jnp.int32),
        ],
    )(e3)


def _sc_scatter_body(t, tk, pad, x_hbm, dest4_hbm, xs_hbm,
                     rows_v0, rows_v1, idx_all, sem0, sem1):
    wid = lax.axis_index("s") * NC + lax.axis_index("c")
    npw = tk // NWORK
    base = wid * npw
    k = base // t
    t0 = base - k * t
    nch = npw // CH_S
    rows = (rows_v0, rows_v1)
    sems = (sem0, sem1)
    pltpu.sync_copy(dest4_hbm.at[wid], idx_all)
    handles = [None, None]
    for c in range(nch):
        bb = c % 2
        if handles[bb] is not None:
            handles[bb].wait()
        off = c * CH_S
        pltpu.sync_copy(x_hbm.at[pl.ds(t0 + off, CH_S)], rows[bb])
        handles[bb] = pltpu.async_copy(rows[bb], xs_hbm.at[idx_all.at[c, 0]],
                                       sems[bb])
    for h in handles:
        if h is not None:
            h.wait()


def _sc_scatter(flat_x, dest4, pad):
    t, d = flat_x.shape
    nw, nch, _, chs = dest4.shape
    tk = nw * nch * chs
    fn = pl.kernel(
        functools.partial(_sc_scatter_body, t, tk, pad),
        out_type=jax.ShapeDtypeStruct((pad, d), jnp.float32),
        mesh=plsc.VectorSubcoreMesh(core_axis_name="c", subcore_axis_name="s"),
        scratch_types=[
            pltpu.VMEM((CH_S, d), jnp.float32),
            pltpu.VMEM((CH_S, d), jnp.float32),
            pltpu.VMEM((nch, 1, CH_S), jnp.int32),
            pltpu.SemaphoreType.DMA,
            pltpu.SemaphoreType.DMA,
        ],
    )
    return fn(flat_x, dest4)


def _expert_body(eot_ref, xs_ref, wd_ref, bd_ref, wu_ref, bu_ref, ys_ref):
    xv = xs_ref[...]
    down = lax.dot_general(xv, wd_ref[0], (((1,), (1,)), ((), ())),
                           preferred_element_type=jnp.float32) + bd_ref[0]
    h = _gelu(down)
    up = lax.dot_general(h, wu_ref[0], (((1,), (1,)), ((), ())),
                         preferred_element_type=jnp.float32) + bu_ref[0]
    ys_ref[...] = up


def _expert_mm(eot, xs, Wd, bd3, Wu, bu3):
    pad, d = xs.shape
    nexp, h = Wd.shape[0], Wd.shape[1]
    nt = pad // TM
    grid_spec = pltpu.PrefetchScalarGridSpec(
        num_scalar_prefetch=1,
        grid=(nt,),
        in_specs=[
            pl.BlockSpec((TM, d), lambda i, eot: (i, 0)),
            pl.BlockSpec((1, h, d), lambda i, eot: (eot[i], 0, 0)),
            pl.BlockSpec((1, 1, h), lambda i, eot: (eot[i], 0, 0)),
            pl.BlockSpec((1, d, h), lambda i, eot: (eot[i], 0, 0)),
            pl.BlockSpec((1, 1, d), lambda i, eot: (eot[i], 0, 0)),
        ],
        out_specs=pl.BlockSpec((TM, d), lambda i, eot: (i, 0)),
    )
    return pl.pallas_call(
        _expert_body,
        grid_spec=grid_spec,
        out_shape=jax.ShapeDtypeStruct((pad, d), jnp.float32),
        compiler_params=pltpu.CompilerParams(
            dimension_semantics=("arbitrary",)),
    )(eot, xs, Wd, bd3, Wu, bu3)


def _sc_combine_body(t, chc, ys_hbm, dest2_hbm, gates_hbm, out_hbm,
                     bufa0, bufa1, bufb0, bufb1, ia_v, ib_v, ga_v, gb_v,
                     sa0, sa1, sb0, sb1):
    d = bufa0.shape[1]
    wid = lax.axis_index("s") * NC + lax.axis_index("c")
    ntw = t // NWORK
    t0 = wid * ntw
    nch = ntw // chc
    bas = (bufa0, bufa1)
    bbs = (bufb0, bufb1)
    sas = (sa0, sa1)
    sbs = (sb0, sb1)

    pltpu.sync_copy(dest2_hbm.at[0, pl.ds(t0, ntw)], ia_v)
    pltpu.sync_copy(dest2_hbm.at[1, pl.ds(t0, ntw)], ib_v)
    pltpu.sync_copy(gates_hbm.at[0, pl.ds(t0, ntw)], ga_v)
    pltpu.sync_copy(gates_hbm.at[1, pl.ds(t0, ntw)], gb_v)

    def issue(c, bb):
        sl = pl.ds(c * chc, chc)
        pltpu.async_copy(ys_hbm.at[ia_v.at[sl]], bas[bb], sas[bb])
        pltpu.async_copy(ys_hbm.at[ib_v.at[sl]], bbs[bb], sbs[bb])

    def drain(bb):
        pltpu.make_async_copy(ys_hbm.at[pl.ds(0, chc)], bas[bb],
                              sas[bb]).wait()
        pltpu.make_async_copy(ys_hbm.at[pl.ds(0, chc)], bbs[bb],
                              sbs[bb]).wait()

    def proc(c, bb):
        ba = bas[bb]
        bbuf = bbs[bb]

        def row(r, _):
            gav = ga_v[c * chc + r, :]
            gbv = gb_v[c * chc + r, :]
            for cc in range(d // 16):
                sll = pl.ds(cc * 16, 16)
                ba[r, sll] = ba[r, sll] * gav + bbuf[r, sll] * gbv
            return 0

        lax.fori_loop(0, chc, row, 0)
        pltpu.sync_copy(ba, out_hbm.at[pl.ds(t0 + c * chc, chc)])

    ngr = nch // 2
    issue(0, 0)

    def group(g, _):
        c0 = 2 * g
        issue(c0 + 1, 1)
        drain(0)
        proc(c0, 0)

        @pl.when(g + 1 < ngr)
        def _():
            issue(c0 + 2, 0)

        drain(1)
        proc(c0 + 1, 1)
        return 0

    lax.fori_loop(0, ngr, group, 0)


def _sc_combine(ys, dest2, gates_rep):
    pad, d = ys.shape
    t = dest2.shape[1]
    ntw = t // NWORK
    chc = min(8, ntw)
    fn = pl.kernel(
        functools.partial(_sc_combine_body, t, chc),
        out_type=jax.ShapeDtypeStruct((t, d), jnp.float32),
        mesh=plsc.VectorSubcoreMesh(core_axis_name="c", subcore_axis_name="s"),
        scratch_types=[
            pltpu.VMEM((chc, d), jnp.float32),
            pltpu.VMEM((chc, d), jnp.float32),
            pltpu.VMEM((chc, d), jnp.float32),
            pltpu.VMEM((chc, d), jnp.float32),
            pltpu.VMEM((ntw,), jnp.int32),
            pltpu.VMEM((ntw,), jnp.int32),
            pltpu.VMEM((ntw, 16), jnp.float32),
            pltpu.VMEM((ntw, 16), jnp.float32),
            pltpu.SemaphoreType.DMA,
            pltpu.SemaphoreType.DMA,
            pltpu.SemaphoreType.DMA,
            pltpu.SemaphoreType.DMA,
        ],
    )
    return fn(ys, dest2, gates_rep)


@jax.jit
def kernel(x, Wg, bg, Wn, bn, Wd, bd, Wu, bu, noise_eps):
    b, s, d = x.shape
    nexp, h = Wd.shape[0], Wd.shape[1]
    t = b * s
    tk = 2 * t
    cj = 128
    rj = tk // cj
    nt = tk // TM + nexp
    pad = nt * TM
    ntc = max(256, nt)

    flat_x = x.reshape(t, d)
    epsT = noise_eps.reshape(t, nexp).T
    wgn = jnp.concatenate([Wg, Wn], axis=0).T
    bgn_col = jnp.concatenate([bg, bn])[:, None]
    bd3 = bd[:, None, :]
    bu3 = bu[:, None, :]

    eids, gates = _router(flat_x, wgn, bgn_col, epsT)
    e3 = eids.reshape(rj, cj)
    dest3, eot2 = _dispatch(e3, nexp, ntc)
    dest4 = dest3.reshape(NWORK, (tk // NWORK) // CH_S, 1, CH_S)
    dest2 = dest3.reshape(2, t)
    eot = eot2.reshape(ntc)

    xs = _sc_scatter(flat_x, dest4, pad)
    ys = _expert_mm(eot, xs, Wd, bd3, Wu, bu3)
    out = _sc_combine(ys, dest2, gates)
    return out.reshape(b, s, d)


# speedup vs baseline: 1.1916x; 1.1916x over previous
import functools

import jax
import jax.numpy as jnp
from jax import lax
from jax.experimental import pallas as pl
from jax.experimental.pallas import tpu as pltpu
from jax.experimental.pallas import tpu_sc as plsc

TM = 256
TBR = 512
CH_S = 16
NC = 2
NWORK = 32


def _softplus(x):
    return jnp.maximum(x, 0.0) + jnp.log1p(jnp.exp(-jnp.abs(x)))


def _gelu(x):
    return x * 0.5 * (1.0 + lax.erf(x * 0.7071067811865476))


def _router_body(x_ref, wgn_ref, bgn_ref, epsT_ref, eid_ref, gate_ref):
    nexp = epsT_ref.shape[0]
    logits2 = lax.dot_general(x_ref[...], wgn_ref[...],
                              (((1,), (0,)), ((), ())),
                              preferred_element_type=jnp.float32)
    l2t = logits2.T + bgn_ref[...]
    lg = l2t[:nexp]
    nl = l2t[nexp:]
    noisy = lg + epsT_ref[...] * _softplus(nl)
    iota = lax.broadcasted_iota(jnp.int32, noisy.shape, 0)
    m1 = jnp.max(noisy, axis=0, keepdims=True)
    a1 = jnp.min(jnp.where(noisy == m1, iota, nexp),
                 axis=0, keepdims=True)
    masked = jnp.where(iota == a1, -jnp.inf, noisy)
    m2 = jnp.max(masked, axis=0, keepdims=True)
    a2 = jnp.min(jnp.where(masked == m2, iota, nexp),
                 axis=0, keepdims=True)
    t = jnp.exp(m2 - m1)
    g1 = 1.0 / (1.0 + t)
    g2 = t * g1
    eid_ref[...] = jnp.concatenate([a1, a2], axis=0).astype(jnp.float32)
    nl16 = gate_ref.shape[2]
    g1b = jnp.broadcast_to(g1[:, :, None], (1, g1.shape[1], nl16))
    g2b = jnp.broadcast_to(g2[:, :, None], (1, g2.shape[1], nl16))
    gate_ref[...] = jnp.concatenate([g1b, g2b], axis=0)


def _router(flat_x, wgn, bgn_col, epsT):
    t, d = flat_x.shape
    nexp = epsT.shape[0]
    grid = (t // TBR,)
    return pl.pallas_call(
        _router_body,
        grid=grid,
        in_specs=[
            pl.BlockSpec((TBR, d), lambda i: (i, 0)),
            pl.BlockSpec((d, 2 * nexp), lambda i: (0, 0)),
            pl.BlockSpec((2 * nexp, 1), lambda i: (0, 0)),
            pl.BlockSpec((nexp, TBR), lambda i: (0, i)),
        ],
        out_specs=[
            pl.BlockSpec((2, TBR), lambda i: (0, i)),
            pl.BlockSpec((2, TBR, 16), lambda i: (0, i, 0)),
        ],
        out_shape=[
            jax.ShapeDtypeStruct((2, t), jnp.float32),
            jax.ShapeDtypeStruct((2, t, 16), jnp.float32),
        ],
        compiler_params=pltpu.CompilerParams(
            dimension_semantics=("parallel",)),
    )(flat_x, wgn, bgn_col, epsT)


def _dispatch_body(nexp, ntc, e3_ref, dest_ref, eot_ref):
    e3 = e3_ref[...].astype(jnp.int32)
    rj, cj = e3.shape
    eio = lax.broadcasted_iota(jnp.int32, (rj, nexp, cj), 1)
    x3 = (e3[:, None, :] == eio).astype(jnp.float32)
    x2 = x3.reshape(rj * nexp, cj)
    ci = lax.broadcasted_iota(jnp.int32, (cj, cj), 0)
    cc = lax.broadcasted_iota(jnp.int32, (cj, cj), 1)
    ltc = (ci <= cc).astype(jnp.float32)
    p2 = lax.dot_general(x2, ltc, (((1,), (0,)), ((), ())),
                         preferred_element_type=jnp.float32)
    p3 = p2.reshape(rj, nexp, cj)
    within = jnp.sum(p3 * x3, axis=1)
    bt = jnp.sum(x3, axis=2)
    ri = lax.broadcasted_iota(jnp.int32, (rj, rj), 0)
    rr = lax.broadcasted_iota(jnp.int32, (rj, rj), 1)
    sl = (rr < ri).astype(jnp.float32)
    exb = lax.dot_general(sl, bt, (((1,), (0,)), ((), ())),
                          preferred_element_type=jnp.float32)
    counts = jnp.sum(bt, axis=0, keepdims=True)
    pe = jnp.ceil(counts / TM) * TM
    ei = lax.broadcasted_iota(jnp.int32, (nexp, nexp), 0)
    ee = lax.broadcasted_iota(jnp.int32, (nexp, nexp), 1)
    sle = (ei < ee).astype(jnp.float32)
    ss = lax.dot_general(pe, sle, (((1,), (0,)), ((), ())),
                         preferred_element_type=jnp.float32)
    base = jnp.sum((exb[:, :, None] + ss[:, :, None]) * x3, axis=1)
    dest_ref[...] = (base + within - 1.0).astype(jnp.int32)
    ti = (lax.broadcasted_iota(jnp.int32, (nexp, ntc), 1) * TM
          ).astype(jnp.float32)
    eot = jnp.sum((ss.T <= ti).astype(jnp.float32), axis=0,
                  keepdims=True) - 1.0
    eot_ref[...] = eot.astype(jnp.int32)


def _dispatch(e3, nexp, ntc):
    rj, cj = e3.shape
    return pl.pallas_call(
        functools.partial(_dispatch_body, nexp, ntc),
        grid=(1,),
        in_specs=[pl.BlockSpec((rj, cj), lambda i: (0, 0))],
        out_specs=[
            pl.BlockSpec((rj, cj), lambda i: (0, 0)),
            pl.BlockSpec((1, ntc), lambda i: (0, 0)),
        ],
        out_shape=[
            jax.ShapeDtypeStruct((rj, cj), jnp.int32),
            jax.ShapeDtypeStruct((1, ntc), jnp.int32),
        ],
    )(e3)


def _sc_scatter_body(t, tk, pad, x_hbm, dest4_hbm, xs_hbm,
                     rows_v0, rows_v1, idx_all, sem0, sem1):
    wid = lax.axis_index("s") * NC + lax.axis_index("c")
    npw = tk // NWORK
    base = wid * npw
    k = base // t
    t0 = base - k * t
    nch = npw // CH_S
    rows = (rows_v0, rows_v1)
    sems = (sem0, sem1)
    pltpu.sync_copy(dest4_hbm.at[wid], idx_all)
    handles = [None, None]
    for c in range(nch):
        bb = c % 2
        if handles[bb] is not None:
            handles[bb].wait()
        off = c * CH_S
        pltpu.sync_copy(x_hbm.at[pl.ds(t0 + off, CH_S)], rows[bb])
        handles[bb] = pltpu.async_copy(rows[bb], xs_hbm.at[idx_all.at[c, 0]],
                                       sems[bb])
    for h in handles:
        if h is not None:
            h.wait()


def _sc_scatter(flat_x, dest4, pad):
    t, d = flat_x.shape
    nw, nch, _, chs = dest4.shape
    tk = nw * nch * chs
    fn = pl.kernel(
        functools.partial(_sc_scatter_body, t, tk, pad),
        out_type=jax.ShapeDtypeStruct((pad, d), jnp.float32),
        mesh=plsc.VectorSubcoreMesh(core_axis_name="c", subcore_axis_name="s"),
        scratch_types=[
            pltpu.VMEM((CH_S, d), jnp.float32),
            pltpu.VMEM((CH_S, d), jnp.float32),
            pltpu.VMEM((nch, 1, CH_S), jnp.int32),
            pltpu.SemaphoreType.DMA,
            pltpu.SemaphoreType.DMA,
        ],
    )
    return fn(flat_x, dest4)


def _expert_body(eot_ref, xs_ref, wd_ref, bd_ref, wu_ref, bu_ref, ys_ref):
    xv = xs_ref[...]
    down = lax.dot_general(xv, wd_ref[0], (((1,), (1,)), ((), ())),
                           preferred_element_type=jnp.float32) + bd_ref[0]
    h = _gelu(down)
    up = lax.dot_general(h, wu_ref[0], (((1,), (1,)), ((), ())),
                         preferred_element_type=jnp.float32) + bu_ref[0]
    ys_ref[...] = up


def _expert_mm(eot, xs, Wd, bd3, Wu, bu3):
    pad, d = xs.shape
    nexp, h = Wd.shape[0], Wd.shape[1]
    nt = pad // TM
    grid_spec = pltpu.PrefetchScalarGridSpec(
        num_scalar_prefetch=1,
        grid=(nt,),
        in_specs=[
            pl.BlockSpec((TM, d), lambda i, eot: (i, 0)),
            pl.BlockSpec((1, h, d), lambda i, eot: (eot[i], 0, 0)),
            pl.BlockSpec((1, 1, h), lambda i, eot: (eot[i], 0, 0)),
            pl.BlockSpec((1, d, h), lambda i, eot: (eot[i], 0, 0)),
            pl.BlockSpec((1, 1, d), lambda i, eot: (eot[i], 0, 0)),
        ],
        out_specs=pl.BlockSpec((TM, d), lambda i, eot: (i, 0)),
    )
    return pl.pallas_call(
        _expert_body,
        grid_spec=grid_spec,
        out_shape=jax.ShapeDtypeStruct((pad, d), jnp.float32),
        compiler_params=pltpu.CompilerParams(
            dimension_semantics=("arbitrary",)),
    )(eot, xs, Wd, bd3, Wu, bu3)


def _sc_combine_body(t, chc, ys_hbm, dest2_hbm, gates_hbm, out_hbm,
                     bufa0, bufa1, bufb0, bufb1, ia_v, ib_v, ga_v, gb_v,
                     sa0, sa1, sb0, sb1):
    d = bufa0.shape[1]
    wid = lax.axis_index("s") * NC + lax.axis_index("c")
    ntw = t // NWORK
    t0 = wid * ntw
    nch = ntw // chc
    bas = (bufa0, bufa1)
    bbs = (bufb0, bufb1)
    sas = (sa0, sa1)
    sbs = (sb0, sb1)

    pltpu.sync_copy(dest2_hbm.at[0, pl.ds(t0, ntw)], ia_v)
    pltpu.sync_copy(dest2_hbm.at[1, pl.ds(t0, ntw)], ib_v)
    pltpu.sync_copy(gates_hbm.at[0, pl.ds(t0, ntw)], ga_v)
    pltpu.sync_copy(gates_hbm.at[1, pl.ds(t0, ntw)], gb_v)

    def issue(c, bb):
        sl = pl.ds(c * chc, chc)
        pltpu.async_copy(ys_hbm.at[ia_v.at[sl]], bas[bb], sas[bb])
        pltpu.async_copy(ys_hbm.at[ib_v.at[sl]], bbs[bb], sbs[bb])

    def drain(bb):
        pltpu.make_async_copy(ys_hbm.at[pl.ds(0, chc)], bas[bb],
                              sas[bb]).wait()
        pltpu.make_async_copy(ys_hbm.at[pl.ds(0, chc)], bbs[bb],
                              sbs[bb]).wait()

    def proc(c, bb):
        ba = bas[bb]
        bbuf = bbs[bb]

        def row(r, _):
            gav = ga_v[c * chc + r, :]
            gbv = gb_v[c * chc + r, :]
            for cc in range(d // 16):
                sll = pl.ds(cc * 16, 16)
                ba[r, sll] = ba[r, sll] * gav + bbuf[r, sll] * gbv
            return 0

        lax.fori_loop(0, chc, row, 0)
        pltpu.sync_copy(ba, out_hbm.at[pl.ds(t0 + c * chc, chc)])

    ngr = nch // 2
    issue(0, 0)

    def group(g, _):
        c0 = 2 * g
        issue(c0 + 1, 1)
        drain(0)
        proc(c0, 0)

        @pl.when(g + 1 < ngr)
        def _():
            issue(c0 + 2, 0)

        drain(1)
        proc(c0 + 1, 1)
        return 0

    lax.fori_loop(0, ngr, group, 0)


def _sc_combine(ys, dest2, gates_rep):
    pad, d = ys.shape
    t = dest2.shape[1]
    ntw = t // NWORK
    chc = min(8, ntw)
    fn = pl.kernel(
        functools.partial(_sc_combine_body, t, chc),
        out_type=jax.ShapeDtypeStruct((t, d), jnp.float32),
        mesh=plsc.VectorSubcoreMesh(core_axis_name="c", subcore_axis_name="s"),
        scratch_types=[
            pltpu.VMEM((chc, d), jnp.float32),
            pltpu.VMEM((chc, d), jnp.float32),
            pltpu.VMEM((chc, d), jnp.float32),
            pltpu.VMEM((chc, d), jnp.float32),
            pltpu.VMEM((ntw,), jnp.int32),
            pltpu.VMEM((ntw,), jnp.int32),
            pltpu.VMEM((ntw, 16), jnp.float32),
            pltpu.VMEM((ntw, 16), jnp.float32),
            pltpu.SemaphoreType.DMA,
            pltpu.SemaphoreType.DMA,
            pltpu.SemaphoreType.DMA,
            pltpu.SemaphoreType.DMA,
        ],
    )
    return fn(ys, dest2, gates_rep)


@jax.jit
def kernel(x, Wg, bg, Wn, bn, Wd, bd, Wu, bu, noise_eps):
    b, s, d = x.shape
    nexp, h = Wd.shape[0], Wd.shape[1]
    t = b * s
    tk = 2 * t
    cj = 128
    rj = tk // cj
    nt = tk // TM + nexp
    pad = nt * TM
    ntc = max(256, nt)

    flat_x = x.reshape(t, d)
    epsT = noise_eps.reshape(t, nexp).T
    wgn = jnp.concatenate([Wg, Wn], axis=0).T
    bgn_col = jnp.concatenate([bg, bn])[:, None]
    bd3 = bd[:, None, :]
    bu3 = bu[:, None, :]

    eids, gates = _router(flat_x, wgn, bgn_col, epsT)
    e3 = eids.reshape(rj, cj)
    dest3, eot2 = _dispatch(e3, nexp, ntc)
    dest4 = dest3.reshape(NWORK, (tk // NWORK) // CH_S, 1, CH_S)
    dest2 = dest3.reshape(2, t)
    eot = eot2.reshape(ntc)

    xs = _sc_scatter(flat_x, dest4, pad)
    ys = _expert_mm(eot, xs, Wd, bd3, Wu, bu3)
    out = _sc_combine(ys, dest2, gates)
    return out.reshape(b, s, d)
